# baseline (speedup 1.0000x reference)
import jax
import jax.numpy as jnp
from jax import lax
from jax.experimental import pallas as pl
from jax.experimental.pallas import tpu as pltpu
from jax.experimental.pallas import tpu_sc as plsc

_V0 = 1.0
_W0 = 1.0
_N = 100000
_E = 6400000

_LANE = 128
_ROWS = 782
_NP = _ROWS * _LANE

_NC = 2
_NS = 16
_NT = _NC * _NS
_K = 10
_CE = _K * _LANE
_NCHUNKS = _E // _CE


def _al(v):
    return pl.multiple_of(v, 8)


def _pre_body(x0_ref, x1_ref, f_ref):
    x0 = x0_ref[...]
    x1 = x1_ref[...]
    f_ref[0] = jnp.cos(x0)
    f_ref[1] = jnp.cos(x1)
    f_ref[2] = jnp.sin(x0)
    f_ref[3] = jnp.sin(x1)


_pre_call = pl.pallas_call(
    _pre_body,
    out_shape=jax.ShapeDtypeStruct((4, _ROWS, _LANE), jnp.float32),
)


def _epi_body(p_ref, t_ref, v0_ref, v1_ref, w_ref):
    tv = t_ref[...]
    ct = jnp.cos(tv)
    st = jnp.sin(tv)
    sc0 = p_ref[0, 0] + p_ref[1, 0]
    sc1 = p_ref[0, 1] + p_ref[1, 1]
    ss0 = p_ref[0, 2] + p_ref[1, 2]
    ss1 = p_ref[0, 3] + p_ref[1, 3]
    m0 = ct * sc0 + st * ss0
    m1 = ct * sc1 + st * ss1
    m2 = ct * ss0 - st * sc0
    m3 = ct * ss1 - st * sc1
    norm = jnp.sqrt(m0 * m0 + m1 * m1 + m2 * m2 + m3 * m3)
    w_ref[...] = _W0 * (m1 / jnp.maximum(norm, 1e-12))
    v0_ref[...] = _V0 * ct
    v1_ref[...] = _V0 * st


_epi_call = pl.pallas_call(
    _epi_body,
    out_shape=(
        jax.ShapeDtypeStruct((_ROWS, _LANE), jnp.float32),
        jax.ShapeDtypeStruct((_ROWS, _LANE), jnp.float32),
        jax.ShapeDtypeStruct((_ROWS, _LANE), jnp.float32),
    ),
)


def _sc_body(ftab_hbm, edges_hbm, zero_hbm, out_hbm,
             src_v, dst_v, sidx_v0, sidx_v1, didx_v0, didx_v1,
             rows_v0, rows_v1, acc,
             sem_a, sem_b, sem_g, sem_s0, sem_s1):
    c = lax.axis_index("c")
    s = lax.axis_index("s")
    wid = c * _NS + s
    sems = (sem_a, sem_b)
    ssems = (sem_s0, sem_s1)
    sidxs = (sidx_v0, sidx_v1)
    didxs = (didx_v0, didx_v1)
    rows = (rows_v0, rows_v1)

    @pl.when(s == 0)
    def _():
        pltpu.sync_copy(zero_hbm, acc)

    plsc.subcore_barrier()

    nq = 156 + jnp.where(wid < _NCHUNKS - 156 * _NT, 1, 0)

    def ebase(gq):
        return _al((wid + _NT * gq) * _CE)

    def drain_scatters(b):
        for j in range(_K):
            pltpu.make_async_copy(rows[b].at[j], acc.at[didxs[b].at[j]],
                                  ssems[b]).wait()

    def process_chunk(b, gq, last):
        eb = ebase(gq)
        pltpu.make_async_copy(edges_hbm.at[pl.ds(eb, _CE)],
                              src_v.at[b], sems[b]).wait()
        pltpu.make_async_copy(edges_hbm.at[pl.ds(_E + eb, _CE)],
                              dst_v.at[b], sems[b]).wait()
        for j in range(_K):
            for l in range(_LANE // 16):
                sl = pl.ds(j * _LANE + l * 16, 16)
                sl2 = pl.ds(l * 16, 16)
                sidxs[b][j, sl2] = src_v[b, sl]
                didxs[b][j, sl2] = dst_v[b, sl]

        @pl.when(gq + 2 <= last)
        def _():
            nb = ebase(gq + 2)
            pltpu.async_copy(edges_hbm.at[pl.ds(nb, _CE)],
                             src_v.at[b], sems[b])
            pltpu.async_copy(edges_hbm.at[pl.ds(_E + nb, _CE)],
                             dst_v.at[b], sems[b])

        @pl.when(gq >= 2)
        def _():
            drain_scatters(b)

        for j in range(_K):
            pltpu.async_copy(ftab_hbm.at[sidxs[b].at[j]],
                             rows[b].at[j], sem_g)
        for j in range(_K):
            pltpu.make_async_copy(ftab_hbm.at[sidxs[b].at[j]],
                                  rows[b].at[j], sem_g).wait()
            pltpu.async_copy(rows[b].at[j], acc.at[didxs[b].at[j]],
                             ssems[b], add=True)

    last = nq - 1
    for b in range(2):
        eb = ebase(b)
        pltpu.async_copy(edges_hbm.at[pl.ds(eb, _CE)], src_v.at[b], sems[b])
        pltpu.async_copy(edges_hbm.at[pl.ds(_E + eb, _CE)],
                         dst_v.at[b], sems[b])

    @pl.loop(0, nq, step=2)
    def _chunks(g):
        for b in range(2):
            gb = g + b

            @pl.when(gb <= last)
            def _():
                process_chunk(b, gb, last)

    drain_scatters(0)
    drain_scatters(1)
    plsc.subcore_barrier()

    @pl.when(s == 0)
    def _():
        pltpu.sync_copy(acc, out_hbm.at[c])


_sc_call = pl.kernel(
    _sc_body,
    out_type=jax.ShapeDtypeStruct((_NC, _NP, 4), jnp.float32),
    mesh=plsc.VectorSubcoreMesh(core_axis_name="c", subcore_axis_name="s"),
    compiler_params=pltpu.CompilerParams(needs_layout_passes=False,
                                         use_tc_tiling_on_sc=False),
    scratch_types=[
        pltpu.VMEM((2, _CE), jnp.int32),
        pltpu.VMEM((2, _CE), jnp.int32),
        pltpu.VMEM((_K, _LANE), jnp.int32),
        pltpu.VMEM((_K, _LANE), jnp.int32),
        pltpu.VMEM((_K, _LANE), jnp.int32),
        pltpu.VMEM((_K, _LANE), jnp.int32),
        pltpu.VMEM((_K, _LANE, 4), jnp.float32),
        pltpu.VMEM((_K, _LANE, 4), jnp.float32),
        pltpu.VMEM_SHARED((_NP, 4), jnp.float32),
        pltpu.SemaphoreType.DMA,
        pltpu.SemaphoreType.DMA,
        pltpu.SemaphoreType.DMA,
        pltpu.SemaphoreType.DMA,
        pltpu.SemaphoreType.DMA,
    ],
)


@jax.jit
def kernel(x, theta, edge_index):
    x = x.astype(jnp.float32)
    theta = theta.astype(jnp.float32)
    pad = _NP - _N
    x0 = jnp.pad(x[:, 0], (0, pad)).reshape(_ROWS, _LANE)
    x1 = jnp.pad(x[:, 1], (0, pad)).reshape(_ROWS, _LANE)
    tp = jnp.pad(theta[:, 0], (0, pad)).reshape(_ROWS, _LANE)

    ftab = _pre_call(x0, x1).reshape(4, _NP).T
    edges = edge_index.reshape(2 * _E)
    zeros = jnp.zeros((_NP, 4), jnp.float32)

    part = _sc_call(ftab, edges, zeros)
    pp = part.transpose(0, 2, 1).reshape(_NC, 4, _ROWS, _LANE)

    v0, v1, w = _epi_call(pp, tp)
    v = jnp.stack([v0.reshape(-1)[:_N], v1.reshape(-1)[:_N]], axis=-1)
    wq = w.reshape(-1)[:_N][:, None]
    return (v, wq)


# speedup vs baseline: 1.1227x; 1.1227x over previous
import jax
import jax.numpy as jnp
from jax import lax
from jax.experimental import pallas as pl
from jax.experimental.pallas import tpu as pltpu
from jax.experimental.pallas import tpu_sc as plsc

_V0 = 1.0
_W0 = 1.0
_N = 100000
_E = 6400000

_LANE = 128
_ROWS = 782
_NP = _ROWS * _LANE

_NC = 2
_NS = 16
_K = 10
_CE = _K * _LANE

_CH_TILES = 8
_CH_EDGES = _E // _CH_TILES
_CH_CHUNKS = _CH_EDGES // _CE


def _al(v):
    return pl.multiple_of(v, 8)


def _pre_body(x0_ref, x1_ref, f_ref):
    x0 = x0_ref[...]
    x1 = x1_ref[...]
    f_ref[0] = jnp.cos(x0)
    f_ref[1] = jnp.cos(x1)
    f_ref[2] = jnp.sin(x0)
    f_ref[3] = jnp.sin(x1)


_pre_call = pl.pallas_call(
    _pre_body,
    out_shape=jax.ShapeDtypeStruct((4, _ROWS, _LANE), jnp.float32),
)


def _epi_body(p_ref, t_ref, v_ref, w_ref):
    tv = t_ref[...]
    ct = jnp.cos(tv)
    st = jnp.sin(tv)
    sc0 = p_ref[0, 0]
    sc1 = p_ref[0, 1]
    ss0 = p_ref[1, 0]
    ss1 = p_ref[1, 1]
    m0 = ct * sc0 + st * ss0
    m1 = ct * sc1 + st * ss1
    m2 = ct * ss0 - st * sc0
    m3 = ct * ss1 - st * sc1
    norm = jnp.sqrt(m0 * m0 + m1 * m1 + m2 * m2 + m3 * m3)
    w_ref[...] = _W0 * (m1 / jnp.maximum(norm, 1e-12))
    vv = jnp.concatenate([(_V0 * ct)[:, :, None], (_V0 * st)[:, :, None]],
                         axis=2)
    v_ref[...] = vv.reshape(_ROWS, 2 * _LANE)


_epi_call = pl.pallas_call(
    _epi_body,
    out_shape=(
        jax.ShapeDtypeStruct((_ROWS, 2 * _LANE), jnp.float32),
        jax.ShapeDtypeStruct((_ROWS, _LANE), jnp.float32),
    ),
)


def _sc_body(ftab_hbm, edges_hbm, zero_hbm, out_hbm,
             ftab_v, src_v, dst_v, idx_v, val_v, acc,
             sem_a, sem_b, sem_add):
    c = lax.axis_index("c")
    s = lax.axis_index("s")
    wid = c * _NS + s
    ch = 2 * c + s // _CH_TILES
    slot = s // _CH_TILES
    rank = s % _CH_TILES
    sems = (sem_a, sem_b)

    pltpu.sync_copy(ftab_hbm.at[pl.ds(_al(ch * _NP), _NP)], ftab_v)

    @pl.when(s == 0)
    def _():
        pltpu.sync_copy(zero_hbm, acc)

    plsc.subcore_barrier()

    choff = slot * _NP

    def process_rows(b, off):
        for j in range(_K):
            for l in range(_LANE // 16):
                sl = pl.ds(j * _LANE + l * 16, 16)
                sl2 = pl.ds(l * 16, 16)
                idx_v[j, sl2] = dst_v[b, sl] + off
                val_v[j, sl2] = plsc.load_gather(ftab_v, [src_v[b, sl]])
            pltpu.async_copy(val_v.at[j], acc.at[idx_v.at[j]],
                             sem_add, add=True)

    def drain_rows():
        for j in range(_K):
            pltpu.make_async_copy(val_v.at[j], acc.at[idx_v.at[j]],
                                  sem_add).wait()

    e0 = rank * _CH_EDGES
    for b in range(2):
        pltpu.async_copy(edges_hbm.at[pl.ds(e0 + b * _CE, _CE)],
                         src_v.at[b], sems[b])
        pltpu.async_copy(edges_hbm.at[pl.ds(_E + e0 + b * _CE, _CE)],
                         dst_v.at[b], sems[b])

    last_ch = _CH_CHUNKS - 1

    @pl.loop(0, _CH_CHUNKS, step=2)
    def _ch_chunks(g):
        for b in range(2):
            gb = g + b

            @pl.when(gb <= last_ch)
            def _():
                rb = _al(e0 + gb * _CE)
                pltpu.make_async_copy(edges_hbm.at[pl.ds(rb, _CE)],
                                      src_v.at[b], sems[b]).wait()
                pltpu.make_async_copy(edges_hbm.at[pl.ds(_E + rb, _CE)],
                                      dst_v.at[b], sems[b]).wait()

                @pl.when(gb > 0)
                def _():
                    drain_rows()

                process_rows(b, choff)

                @pl.when(gb + 2 <= last_ch)
                def _():
                    nb = _al(e0 + (gb + 2) * _CE)
                    pltpu.async_copy(edges_hbm.at[pl.ds(nb, _CE)],
                                     src_v.at[b], sems[b])
                    pltpu.async_copy(edges_hbm.at[pl.ds(_E + nb, _CE)],
                                     dst_v.at[b], sems[b])

    drain_rows()
    plsc.subcore_barrier()

    @pl.when(s == 0)
    def _():
        pltpu.sync_copy(acc, out_hbm.at[pl.ds(_al(c * 2 * _NP), 2 * _NP)])


_sc_call = pl.kernel(
    _sc_body,
    out_type=jax.ShapeDtypeStruct((_NC * 2 * _NP,), jnp.float32),
    mesh=plsc.VectorSubcoreMesh(core_axis_name="c", subcore_axis_name="s"),
    compiler_params=pltpu.CompilerParams(needs_layout_passes=False),
    scratch_types=[
        pltpu.VMEM((_NP,), jnp.float32),
        pltpu.VMEM((2, _CE), jnp.int32),
        pltpu.VMEM((2, _CE), jnp.int32),
        pltpu.VMEM((_K, _LANE), jnp.int32),
        pltpu.VMEM((_K, _LANE), jnp.float32),
        pltpu.VMEM_SHARED((2 * _NP,), jnp.float32),
        pltpu.SemaphoreType.DMA,
        pltpu.SemaphoreType.DMA,
        pltpu.SemaphoreType.DMA,
    ],
)


@jax.jit
def kernel(x, theta, edge_index):
    x = x.astype(jnp.float32)
    theta = theta.astype(jnp.float32)
    pad = _NP - _N
    x0 = jnp.pad(x[:, 0], (0, pad)).reshape(_ROWS, _LANE)
    x1 = jnp.pad(x[:, 1], (0, pad)).reshape(_ROWS, _LANE)
    tp = jnp.pad(theta[:, 0], (0, pad)).reshape(_ROWS, _LANE)

    ftab = _pre_call(x0, x1).reshape(4 * _NP)
    edges = edge_index.reshape(2 * _E)
    zeros = jnp.zeros((2 * _NP,), jnp.float32)

    part = _sc_call(ftab, edges, zeros)
    pp = part.reshape(_NC, 2, _ROWS, _LANE)

    vpairs, w = _epi_call(pp, tp)
    v = vpairs.reshape(-1)[:2 * _N].reshape(_N, 2)
    wq = w.reshape(-1)[:_N][:, None]
    return (v, wq)


# speedup vs baseline: 1.4856x; 1.3233x over previous
import jax
import jax.numpy as jnp
from jax import lax
from jax.experimental import pallas as pl
from jax.experimental.pallas import tpu as pltpu
from jax.experimental.pallas import tpu_sc as plsc

_V0 = 1.0
_W0 = 1.0
_N = 100000
_E = 6400000

_LANE = 128
_ROWS = 782
_NP = _ROWS * _LANE

_NC = 2
_NS = 16
_K = 10
_CE = _K * _LANE

_CH_TILES = 8
_CH_EDGES = _E // _CH_TILES
_CH_CHUNKS = _CH_EDGES // _CE


def _al(v):
    return pl.multiple_of(v, 8)


def _pre_body(x0_ref, x1_ref, f_ref):
    x0 = x0_ref[...]
    x1 = x1_ref[...]
    f_ref[0] = jnp.cos(x0)
    f_ref[1] = jnp.cos(x1)
    f_ref[2] = jnp.sin(x0)
    f_ref[3] = jnp.sin(x1)


_pre_call = pl.pallas_call(
    _pre_body,
    out_shape=jax.ShapeDtypeStruct((4, _ROWS, _LANE), jnp.float32),
)


def _epi_body(p_ref, t_ref, v0_ref, v1_ref, w_ref):
    tv = t_ref[...]
    ct = jnp.cos(tv)
    st = jnp.sin(tv)
    sc0 = p_ref[0, 0]
    sc1 = p_ref[0, 1]
    ss0 = p_ref[1, 0]
    ss1 = p_ref[1, 1]
    m0 = ct * sc0 + st * ss0
    m1 = ct * sc1 + st * ss1
    m2 = ct * ss0 - st * sc0
    m3 = ct * ss1 - st * sc1
    norm = jnp.sqrt(m0 * m0 + m1 * m1 + m2 * m2 + m3 * m3)
    w_ref[...] = _W0 * (m1 / jnp.maximum(norm, 1e-12))
    v0_ref[...] = _V0 * ct
    v1_ref[...] = _V0 * st


_epi_call = pl.pallas_call(
    _epi_body,
    out_shape=(
        jax.ShapeDtypeStruct((_ROWS, _LANE), jnp.float32),
        jax.ShapeDtypeStruct((_ROWS, _LANE), jnp.float32),
        jax.ShapeDtypeStruct((_ROWS, _LANE), jnp.float32),
    ),
)


def _sc_body(ftab_hbm, edges_hbm, zero_hbm, out_hbm,
             ftab_v, src_v, dst_v, idx_v, val_v, acc,
             sem_a, sem_b, sem_add):
    c = lax.axis_index("c")
    s = lax.axis_index("s")
    wid = c * _NS + s
    ch = 2 * c + s // _CH_TILES
    slot = s // _CH_TILES
    rank = s % _CH_TILES
    sems = (sem_a, sem_b)

    pltpu.sync_copy(ftab_hbm.at[pl.ds(_al(ch * _NP), _NP)], ftab_v)

    @pl.when(s == 0)
    def _():
        pltpu.sync_copy(zero_hbm, acc)

    plsc.subcore_barrier()

    choff = slot * _NP

    def process_rows(b, off):
        for j in range(_K):
            for l in range(_LANE // 16):
                sl = pl.ds(j * _LANE + l * 16, 16)
                sl2 = pl.ds(l * 16, 16)
                idx_v[j, sl2] = dst_v[b, sl] + off
                val_v[j, sl2] = plsc.load_gather(ftab_v, [src_v[b, sl]])
            pltpu.async_copy(val_v.at[j], acc.at[idx_v.at[j]],
                             sem_add, add=True)

    def drain_rows():
        for j in range(_K):
            pltpu.make_async_copy(val_v.at[j], acc.at[idx_v.at[j]],
                                  sem_add).wait()

    e0 = rank * _CH_EDGES
    for b in range(2):
        pltpu.async_copy(edges_hbm.at[pl.ds(e0 + b * _CE, _CE)],
                         src_v.at[b], sems[b])
        pltpu.async_copy(edges_hbm.at[pl.ds(_E + e0 + b * _CE, _CE)],
                         dst_v.at[b], sems[b])

    last_ch = _CH_CHUNKS - 1

    @pl.loop(0, _CH_CHUNKS, step=2)
    def _ch_chunks(g):
        for b in range(2):
            gb = g + b

            @pl.when(gb <= last_ch)
            def _():
                rb = _al(e0 + gb * _CE)
                pltpu.make_async_copy(edges_hbm.at[pl.ds(rb, _CE)],
                                      src_v.at[b], sems[b]).wait()
                pltpu.make_async_copy(edges_hbm.at[pl.ds(_E + rb, _CE)],
                                      dst_v.at[b], sems[b]).wait()

                @pl.when(gb > 0)
                def _():
                    drain_rows()

                process_rows(b, choff)

                @pl.when(gb + 2 <= last_ch)
                def _():
                    nb = _al(e0 + (gb + 2) * _CE)
                    pltpu.async_copy(edges_hbm.at[pl.ds(nb, _CE)],
                                     src_v.at[b], sems[b])
                    pltpu.async_copy(edges_hbm.at[pl.ds(_E + nb, _CE)],
                                     dst_v.at[b], sems[b])

    drain_rows()
    plsc.subcore_barrier()

    @pl.when(s == 0)
    def _():
        pltpu.sync_copy(acc, out_hbm.at[pl.ds(_al(c * 2 * _NP), 2 * _NP)])


_sc_call = pl.kernel(
    _sc_body,
    out_type=jax.ShapeDtypeStruct((_NC * 2 * _NP,), jnp.float32),
    mesh=plsc.VectorSubcoreMesh(core_axis_name="c", subcore_axis_name="s"),
    compiler_params=pltpu.CompilerParams(needs_layout_passes=False),
    scratch_types=[
        pltpu.VMEM((_NP,), jnp.float32),
        pltpu.VMEM((2, _CE), jnp.int32),
        pltpu.VMEM((2, _CE), jnp.int32),
        pltpu.VMEM((_K, _LANE), jnp.int32),
        pltpu.VMEM((_K, _LANE), jnp.float32),
        pltpu.VMEM_SHARED((2 * _NP,), jnp.float32),
        pltpu.SemaphoreType.DMA,
        pltpu.SemaphoreType.DMA,
        pltpu.SemaphoreType.DMA,
    ],
)


@jax.jit
def kernel(x, theta, edge_index):
    x = x.astype(jnp.float32)
    theta = theta.astype(jnp.float32)
    pad = _NP - _N
    x0 = jnp.pad(x[:, 0], (0, pad)).reshape(_ROWS, _LANE)
    x1 = jnp.pad(x[:, 1], (0, pad)).reshape(_ROWS, _LANE)
    tp = jnp.pad(theta[:, 0], (0, pad)).reshape(_ROWS, _LANE)

    ftab = _pre_call(x0, x1).reshape(4 * _NP)
    edges = edge_index.reshape(2 * _E)
    zeros = jnp.zeros((2 * _NP,), jnp.float32)

    part = _sc_call(ftab, edges, zeros)
    pp = part.reshape(_NC, 2, _ROWS, _LANE)

    v0, v1, w = _epi_call(pp, tp)
    v = jnp.stack([v0.reshape(-1)[:_N], v1.reshape(-1)[:_N]], axis=-1)
    wq = w.reshape(-1)[:_N][:, None]
    return (v, wq)
